# baseline (speedup 1.0000x reference)
import jax
import jax.numpy as jnp
from jax.experimental import pallas as pl
from jax.experimental.pallas import tpu as pltpu

_NPAD = 102400
_BLOCK = 102400
_SUB = 2048
_PAD_COORD = 100.0


def _distnet_kernel(xt_ref, ptst_ref, beta_ref, out_ref):
    i = pl.program_id(0)
    n = pl.num_programs(0)
    xt = xt_ref[...]
    x = xt.T
    pts_t = ptst_ref[...]
    ptf = pts_t.astype(jnp.float32)
    pp = jnp.sum(ptf * ptf, axis=0, keepdims=True)
    lhs = jnp.concatenate(
        [jnp.ones((x.shape[0], 1), jnp.bfloat16), x.astype(jnp.bfloat16)],
        axis=1)
    rhs = jnp.concatenate(
        [pp.astype(jnp.bfloat16), -2.0 * pts_t], axis=0)
    mblk = None
    prev = None
    nb = ptst_ref.shape[1]
    for s in range(0, nb, _SUB):
        partial = jax.lax.dot_general(
            lhs, rhs[:, s:min(s + _SUB, nb)], (((1,), (0,)), ((), ())),
            preferred_element_type=jnp.float32)
        if prev is not None:
            m = jnp.min(prev, axis=1, keepdims=True)
            mblk = m if mblk is None else jnp.minimum(mblk, m)
        prev = partial
    m = jnp.min(prev, axis=1, keepdims=True)
    mblk = m if mblk is None else jnp.minimum(mblk, m)

    @pl.when(i == 0)
    def _first():
        out_ref[...] = mblk

    @pl.when(i > 0)
    def _acc():
        out_ref[...] = jnp.minimum(out_ref[...], mblk)

    @pl.when(i == n - 1)
    def _fin():
        xx = jnp.sum(x * x, axis=1, keepdims=True)
        d = jnp.maximum(out_ref[...] + xx, 0.0)
        b = jax.nn.softplus(beta_ref[0, 0])
        alpha = -b * 6.9077542789816375
        out_ref[...] = jax.nn.sigmoid((d + alpha) / b)


def kernel(x, points, beta):
    q, dim = x.shape
    n_pts = points.shape[0]
    pts_t = points.T.astype(jnp.bfloat16)
    xt = x.T
    beta2d = beta.reshape(1, 1)
    n_steps = 1
    out = pl.pallas_call(
        _distnet_kernel,
        grid=(n_steps,),
        in_specs=[
            pl.BlockSpec((dim, q), lambda i: (0, 0)),
            pl.BlockSpec((dim, n_pts), lambda i: (0, i)),
            pl.BlockSpec((1, 1), lambda i: (0, 0)),
        ],
        out_specs=pl.BlockSpec((q, 1), lambda i: (0, 0)),
        out_shape=jax.ShapeDtypeStruct((q, 1), jnp.float32),
    )(xt, pts_t, beta2d)
    return out.reshape(q)


# speedup vs baseline: 1.0121x; 1.0121x over previous
import jax
import jax.numpy as jnp
from jax.experimental import pallas as pl
from jax.experimental.pallas import tpu as pltpu

_NPAD = 102400
_BLOCK = 102400
_SUB = 4096
_PAD_COORD = 100.0


def _distnet_kernel(xt_ref, ptst_ref, beta_ref, out_ref):
    i = pl.program_id(0)
    n = pl.num_programs(0)
    xt = xt_ref[...]
    x = xt.T
    pts_t = ptst_ref[...]
    ptf = pts_t.astype(jnp.float32)
    pp = jnp.sum(ptf * ptf, axis=0, keepdims=True)
    lhs = jnp.concatenate(
        [jnp.ones((x.shape[0], 1), jnp.bfloat16), x.astype(jnp.bfloat16)],
        axis=1)
    rhs = jnp.concatenate(
        [pp.astype(jnp.bfloat16), -2.0 * pts_t], axis=0)
    mblk = None
    prev = None
    nb = ptst_ref.shape[1]
    for s in range(0, nb, _SUB):
        partial = jax.lax.dot_general(
            lhs, rhs[:, s:min(s + _SUB, nb)], (((1,), (0,)), ((), ())),
            preferred_element_type=jnp.float32)
        if prev is not None:
            m = jnp.min(prev, axis=1, keepdims=True)
            mblk = m if mblk is None else jnp.minimum(mblk, m)
        prev = partial
    m = jnp.min(prev, axis=1, keepdims=True)
    mblk = m if mblk is None else jnp.minimum(mblk, m)

    @pl.when(i == 0)
    def _first():
        out_ref[...] = mblk

    @pl.when(i > 0)
    def _acc():
        out_ref[...] = jnp.minimum(out_ref[...], mblk)

    @pl.when(i == n - 1)
    def _fin():
        xx = jnp.sum(x * x, axis=1, keepdims=True)
        d = jnp.maximum(out_ref[...] + xx, 0.0)
        b = jax.nn.softplus(beta_ref[0, 0])
        alpha = -b * 6.9077542789816375
        out_ref[...] = jax.nn.sigmoid((d + alpha) / b)


def kernel(x, points, beta):
    q, dim = x.shape
    n_pts = points.shape[0]
    pts_t = points.T.astype(jnp.bfloat16)
    xt = x.T
    beta2d = beta.reshape(1, 1)
    n_steps = 1
    out = pl.pallas_call(
        _distnet_kernel,
        grid=(n_steps,),
        in_specs=[
            pl.BlockSpec((dim, q), lambda i: (0, 0)),
            pl.BlockSpec((dim, n_pts), lambda i: (0, i)),
            pl.BlockSpec((1, 1), lambda i: (0, 0)),
        ],
        out_specs=pl.BlockSpec((q, 1), lambda i: (0, 0)),
        out_shape=jax.ShapeDtypeStruct((q, 1), jnp.float32),
    )(xt, pts_t, beta2d)
    return out.reshape(q)
